# baseline (speedup 1.0000x reference)
import functools

import jax
import jax.numpy as jnp
from jax import lax
from jax.experimental import pallas as pl
from jax.experimental.pallas import tpu as pltpu
from jax.experimental.pallas import tpu_sc as plsc

T_HIST = 50
T_FUT = 60
T_TOT = T_HIST + T_FUT
TWO_T = 2 * T_TOT
D_ENC = 512
MODES6 = 6
HIDDEN = 4096
HALF = HIDDEN // 2
CHUNK = 512
GRID = HIDDEN // CHUNK

_TN = (((0,), (0,)), ((), ()))
_TT = (((0,), (1,)), ((), ()))


def _sc_gather(idx_hbm, pos_hbm, vel_hbm, head_hbm,
               pos_out, vel_out, head_out, idx_s):
    @pl.when(lax.axis_index("c") == 0)
    def _gather():
        pltpu.sync_copy(idx_hbm, idx_s)
        i = idx_s[0]
        pltpu.sync_copy(pos_hbm.at[i], pos_out)
        pltpu.sync_copy(vel_hbm.at[i], vel_out)
        pltpu.sync_copy(head_hbm.at[pl.ds(i, 1)], head_out)


def _gather_rows(idx, position, velocity, heading):
    mesh = plsc.ScalarSubcoreMesh(axis_name="c", num_cores=2)
    f32 = jnp.float32
    return pl.kernel(
        _sc_gather,
        out_type=[
            jax.ShapeDtypeStruct((T_TOT, 3), f32),
            jax.ShapeDtypeStruct((T_TOT, 3), f32),
            jax.ShapeDtypeStruct((1, T_TOT), f32),
        ],
        mesh=mesh,
        scratch_types=[
            pltpu.SMEM((1,), jnp.int32),
        ],
    )(idx, position, velocity, heading)


def _policy_kernel(states_ref, pos_ref, vel_ref, head_ref,
                   wenc_ref, wdp_ref, wdh_ref,
                   w1_ref, b1_ref, w2_ref, b2_ref, w3_ref, b3_ref,
                   mean_ref, std_ref, act_s, acc_s):
    g = pl.program_id(0)

    @pl.when(g == 0)
    def _build_action():
        x_col = pos_ref[:, 0:1]
        y_col = pos_ref[:, 1:2]
        st = states_ref[...]
        st_pad = jnp.pad(st, ((T_HIST, 0), (0, 0)))
        t110 = jax.lax.broadcasted_iota(jnp.int32, (T_TOT, 1), 0)
        fut = t110 >= T_HIST
        x_col = jnp.where(fut, st_pad[:, 0:1], x_col)
        y_col = jnp.where(fut, st_pad[:, 1:2], y_col)
        r220 = jax.lax.broadcasted_iota(jnp.int32, (TWO_T, T_TOT), 0)
        c110 = jax.lax.broadcasted_iota(jnp.int32, (TWO_T, T_TOT), 1)
        px = (r220 == 2 * c110).astype(jnp.float32)
        py = (r220 == 2 * c110 + 1).astype(jnp.float32)
        xy = (jnp.dot(px, x_col, preferred_element_type=jnp.float32)
              + jnp.dot(py, y_col, preferred_element_type=jnp.float32))
        feat = jnp.tanh(jax.lax.dot_general(
            xy, wenc_ref[...], _TN, preferred_element_type=jnp.float32))
        dec = jnp.dot(feat, wdp_ref[...],
                      preferred_element_type=jnp.float32)
        dech = jnp.dot(feat, wdh_ref[...],
                       preferred_element_type=jnp.float32)
        rp = jax.lax.broadcasted_iota(jnp.int32, (MODES6 * T_FUT * 2, T_FUT), 0)
        cp = jax.lax.broadcasted_iota(jnp.int32, (MODES6 * T_FUT * 2, T_FUT), 1)
        sel_x = (rp == 2 * cp).astype(jnp.float32)
        sel_y = (rp == 2 * cp + 1).astype(jnp.float32)
        rh = jax.lax.broadcasted_iota(jnp.int32, (MODES6 * T_FUT, T_FUT), 0)
        ch = jax.lax.broadcasted_iota(jnp.int32, (MODES6 * T_FUT, T_FUT), 1)
        sel_h = (rh == ch).astype(jnp.float32)
        lpx = jax.lax.dot_general(sel_x, dec, _TT,
                                  preferred_element_type=jnp.float32)
        lpy = jax.lax.dot_general(sel_y, dec, _TT,
                                  preferred_element_type=jnp.float32)
        lh = jax.lax.dot_general(sel_h, dech, _TT,
                                 preferred_element_type=jnp.float32)
        theta = head_ref[0, T_HIST - 1]
        c, s = jnp.cos(theta), jnp.sin(theta)
        ox = pos_ref[T_HIST - 1, 0]
        oy = pos_ref[T_HIST - 1, 1]
        npx = lpx * c - lpy * s + ox
        npy = lpx * s + lpy * c + oy
        t60 = jax.lax.broadcasted_iota(jnp.int32, (T_FUT, 1), 0)
        z1 = jnp.zeros((1, 1), jnp.float32)
        px_prev = jnp.concatenate([z1, npx[:-1, :]], axis=0)
        py_prev = jnp.concatenate([z1, npy[:-1, :]], axis=0)
        vx = jnp.where(t60 == 0, npx - ox, (npx - px_prev) * 10.0)
        vy = jnp.where(t60 == 0, npy - oy, (npy - py_prev) * 10.0)
        v49x = vel_ref[T_HIST - 1, 0]
        v49y = vel_ref[T_HIST - 1, 1]
        vx_prev = jnp.concatenate([z1, vx[:-1, :]], axis=0)
        vy_prev = jnp.concatenate([z1, vy[:-1, :]], axis=0)
        ax = jnp.where(t60 == 0, vx - v49x, (vx - vx_prev) * 10.0)
        ay = jnp.where(t60 == 0, vy - v49y, (vy - vy_prev) * 10.0)
        two_pi = 2.0 * jnp.pi
        hd = lh + theta
        hd = (hd + jnp.pi) % two_pi - jnp.pi
        act_s[...] = jnp.concatenate([hd, ax, ay], axis=1)
        acc_s[...] = jnp.zeros_like(acc_s)

    hd = act_s[:, 0:1]
    ax = act_s[:, 1:2]
    ay = act_s[:, 2:3]
    h1 = (hd * w1_ref[0:1, :] + ax * w1_ref[1:2, :] + ay * w1_ref[2:3, :]
          + b1_ref[...])
    h1 = jnp.maximum(h1, 0.0)
    acc_s[...] += jnp.dot(h1, w2_ref[...], preferred_element_type=jnp.float32)

    @pl.when(g == GRID - 1)
    def _finalize():
        h2 = acc_s[...] + b2_ref[...]
        y = jnp.dot(h2, w3_ref[...], preferred_element_type=jnp.float32) \
            + b3_ref[...]
        mean_ref[...] = jnp.tanh(y)
        std_ref[...] = jnp.log1p(jnp.exp(-jnp.abs(y))) + jnp.maximum(y, 0.0) \
            + 1e-8


def kernel(states, position, velocity, heading, predict_mask, agent_index,
           W_enc, W_dec_pos, W_dec_head, W1, b1, W2, b2, W3, b3):
    del predict_mask
    idx = jnp.asarray(agent_index, jnp.int32).reshape((1,))
    pos_row, vel_row, head_row = _gather_rows(idx, position, velocity, heading)
    b1r = b1.reshape(1, HIDDEN)
    b2r = b2.reshape(1, HALF)
    b3r = b3.reshape(1, 3)

    mean, std = pl.pallas_call(
        _policy_kernel,
        grid=(GRID,),
        in_specs=[
            pl.BlockSpec((T_FUT, 4), lambda g: (0, 0)),
            pl.BlockSpec((T_TOT, 3), lambda g: (0, 0)),
            pl.BlockSpec((T_TOT, 3), lambda g: (0, 0)),
            pl.BlockSpec((1, T_TOT), lambda g: (0, 0)),
            pl.BlockSpec((TWO_T, D_ENC), lambda g: (0, 0)),
            pl.BlockSpec((D_ENC, MODES6 * T_FUT * 2), lambda g: (0, 0)),
            pl.BlockSpec((D_ENC, MODES6 * T_FUT), lambda g: (0, 0)),
            pl.BlockSpec((3, CHUNK), lambda g: (0, g)),
            pl.BlockSpec((1, CHUNK), lambda g: (0, g)),
            pl.BlockSpec((CHUNK, HALF), lambda g: (g, 0)),
            pl.BlockSpec((1, HALF), lambda g: (0, 0)),
            pl.BlockSpec((HALF, 3), lambda g: (0, 0)),
            pl.BlockSpec((1, 3), lambda g: (0, 0)),
        ],
        out_specs=[
            pl.BlockSpec((T_FUT, 3), lambda g: (0, 0)),
            pl.BlockSpec((T_FUT, 3), lambda g: (0, 0)),
        ],
        scratch_shapes=[
            pltpu.VMEM((T_FUT, 3), jnp.float32),
            pltpu.VMEM((T_FUT, HALF), jnp.float32),
        ],
        out_shape=[
            jax.ShapeDtypeStruct((T_FUT, 3), jnp.float32),
            jax.ShapeDtypeStruct((T_FUT, 3), jnp.float32),
        ],
    )(states, pos_row, vel_row, head_row,
      W_enc, W_dec_pos, W_dec_head,
      W1, b1r, W2, b2r, W3, b3r)
    return (mean, std)


# speedup vs baseline: 13.1856x; 13.1856x over previous
import jax
import jax.numpy as jnp
from jax import lax
from jax.experimental import pallas as pl
from jax.experimental.pallas import tpu as pltpu
from jax.experimental.pallas import tpu_sc as plsc

T_HIST = 50
T_FUT = 60
T_TOT = T_HIST + T_FUT
TWO_T = 2 * T_TOT
D_ENC = 512
MODES6 = 6
HIDDEN = 4096
HALF = HIDDEN // 2
CHUNK = 512
GRID = HIDDEN // CHUNK
STRIPE = 128
TSUB = 48

_TN = (((0,), (0,)), ((), ()))
_TT = (((0,), (1,)), ((), ()))


def _sc_gather(idx_hbm, pos_hbm, vel_hbm, head_hbm,
               pos_out, vel_out, head_out, idx_s):
    @pl.when(lax.axis_index("c") == 0)
    def _gather():
        pltpu.sync_copy(idx_hbm, idx_s)
        ib = (idx_s[0] // STRIPE) * STRIPE
        pltpu.sync_copy(
            pos_hbm.at[pl.ds(0, 2), :, pl.ds(ib, STRIPE)], pos_out)
        pltpu.sync_copy(
            vel_hbm.at[pl.ds(0, 2), pl.ds(TSUB, 8), pl.ds(ib, STRIPE)],
            vel_out)
        pltpu.sync_copy(head_hbm.at[pl.ds(TSUB, 8), pl.ds(ib, STRIPE)],
                        head_out)


def _gather_stripes(idx, pos_t, vel_t, head_t):
    mesh = plsc.ScalarSubcoreMesh(axis_name="c", num_cores=2)
    f32 = jnp.float32
    return pl.kernel(
        _sc_gather,
        out_type=[
            jax.ShapeDtypeStruct((2, T_TOT, STRIPE), f32),
            jax.ShapeDtypeStruct((2, 8, STRIPE), f32),
            jax.ShapeDtypeStruct((8, STRIPE), f32),
        ],
        mesh=mesh,
        scratch_types=[
            pltpu.SMEM((1,), jnp.int32),
        ],
    )(idx, pos_t, vel_t, head_t)


def _policy_kernel(idx_ref, states_ref, pos8_ref, vel8_ref, head8_ref,
                   wenc_ref, wdp_ref, wdh_ref,
                   w1_ref, b1_ref, w2_ref, b2_ref, w3_ref, b3_ref,
                   mean_ref, std_ref, act_s, acc_s):
    g = pl.program_id(0)

    @pl.when(g == 0)
    def _build_action():
        r = jnp.remainder(idx_ref[0], STRIPE)
        lane_p = jax.lax.broadcasted_iota(jnp.int32, (TWO_T, STRIPE), 1)
        cols = jnp.sum(jnp.where(lane_p == r, pos8_ref[...], 0.0),
                       axis=1, keepdims=True)
        x_col = cols[0:T_TOT, :]
        y_col = cols[T_TOT:TWO_T, :]
        lane_v = jax.lax.broadcasted_iota(jnp.int32, (16, STRIPE), 1)
        v49c = jnp.sum(jnp.where(lane_v == r, vel8_ref[...], 0.0),
                       axis=1, keepdims=True)
        lane_h = jax.lax.broadcasted_iota(jnp.int32, (8, STRIPE), 1)
        thc = jnp.sum(jnp.where(lane_h == r, head8_ref[...], 0.0),
                      axis=1, keepdims=True)
        theta = thc[T_HIST - 1 - TSUB, 0]
        ox = x_col[T_HIST - 1, 0]
        oy = y_col[T_HIST - 1, 0]
        v49x = v49c[(T_HIST - 1 - TSUB), 0]
        v49y = v49c[8 + (T_HIST - 1 - TSUB), 0]
        st = states_ref[...]
        st_pad = jnp.pad(st, ((T_HIST, 0), (0, 0)))
        t110 = jax.lax.broadcasted_iota(jnp.int32, (T_TOT, 1), 0)
        fut = t110 >= T_HIST
        x_col = jnp.where(fut, st_pad[:, 0:1], x_col)
        y_col = jnp.where(fut, st_pad[:, 1:2], y_col)
        r220 = jax.lax.broadcasted_iota(jnp.int32, (TWO_T, T_TOT), 0)
        c110 = jax.lax.broadcasted_iota(jnp.int32, (TWO_T, T_TOT), 1)
        px = (r220 == 2 * c110).astype(jnp.float32)
        py = (r220 == 2 * c110 + 1).astype(jnp.float32)
        xy = (jnp.dot(px, x_col, preferred_element_type=jnp.float32)
              + jnp.dot(py, y_col, preferred_element_type=jnp.float32))
        feat = jnp.tanh(jax.lax.dot_general(
            xy, wenc_ref[...], _TN, preferred_element_type=jnp.float32))
        dec = jnp.dot(feat, wdp_ref[...],
                      preferred_element_type=jnp.float32)
        dech = jnp.dot(feat, wdh_ref[...],
                       preferred_element_type=jnp.float32)
        rp = jax.lax.broadcasted_iota(jnp.int32, (MODES6 * T_FUT * 2, T_FUT), 0)
        cp = jax.lax.broadcasted_iota(jnp.int32, (MODES6 * T_FUT * 2, T_FUT), 1)
        sel_x = (rp == 2 * cp).astype(jnp.float32)
        sel_y = (rp == 2 * cp + 1).astype(jnp.float32)
        rh = jax.lax.broadcasted_iota(jnp.int32, (MODES6 * T_FUT, T_FUT), 0)
        ch = jax.lax.broadcasted_iota(jnp.int32, (MODES6 * T_FUT, T_FUT), 1)
        sel_h = (rh == ch).astype(jnp.float32)
        lpx = jax.lax.dot_general(sel_x, dec, _TT,
                                  preferred_element_type=jnp.float32)
        lpy = jax.lax.dot_general(sel_y, dec, _TT,
                                  preferred_element_type=jnp.float32)
        lh = jax.lax.dot_general(sel_h, dech, _TT,
                                 preferred_element_type=jnp.float32)
        c, s = jnp.cos(theta), jnp.sin(theta)
        npx = lpx * c - lpy * s + ox
        npy = lpx * s + lpy * c + oy
        t60 = jax.lax.broadcasted_iota(jnp.int32, (T_FUT, 1), 0)
        z1 = jnp.zeros((1, 1), jnp.float32)
        px_prev = jnp.concatenate([z1, npx[:-1, :]], axis=0)
        py_prev = jnp.concatenate([z1, npy[:-1, :]], axis=0)
        vx = jnp.where(t60 == 0, npx - ox, (npx - px_prev) * 10.0)
        vy = jnp.where(t60 == 0, npy - oy, (npy - py_prev) * 10.0)
        vx_prev = jnp.concatenate([z1, vx[:-1, :]], axis=0)
        vy_prev = jnp.concatenate([z1, vy[:-1, :]], axis=0)
        ax = jnp.where(t60 == 0, vx - v49x, (vx - vx_prev) * 10.0)
        ay = jnp.where(t60 == 0, vy - v49y, (vy - vy_prev) * 10.0)
        two_pi = 2.0 * jnp.pi
        hd = lh + theta
        hd = (hd + jnp.pi) % two_pi - jnp.pi
        act_s[...] = jnp.concatenate([hd, ax, ay], axis=1)
        acc_s[...] = jnp.zeros_like(acc_s)

    hd = act_s[:, 0:1]
    ax = act_s[:, 1:2]
    ay = act_s[:, 2:3]
    h1 = (hd * w1_ref[0:1, :] + ax * w1_ref[1:2, :] + ay * w1_ref[2:3, :]
          + b1_ref[...])
    h1 = jnp.maximum(h1, 0.0)
    acc_s[...] += jnp.dot(h1, w2_ref[...], preferred_element_type=jnp.float32)

    @pl.when(g == GRID - 1)
    def _finalize():
        h2 = acc_s[...] + b2_ref[...]
        y = jnp.dot(h2, w3_ref[...], preferred_element_type=jnp.float32) \
            + b3_ref[...]
        mean_ref[...] = jnp.tanh(y)
        std_ref[...] = jnp.log1p(jnp.exp(-jnp.abs(y))) + jnp.maximum(y, 0.0) \
            + 1e-8


def kernel(states, position, velocity, heading, predict_mask, agent_index,
           W_enc, W_dec_pos, W_dec_head, W1, b1, W2, b2, W3, b3):
    del predict_mask
    idx = jnp.asarray(agent_index, jnp.int32).reshape((1,))
    pos_t = jnp.transpose(position, (2, 1, 0))
    vel_t = jnp.transpose(velocity, (2, 1, 0))
    head_t = jnp.transpose(heading, (1, 0))
    pos8, vel8, head8 = _gather_stripes(idx, pos_t, vel_t, head_t)
    pos8 = pos8.reshape(TWO_T, STRIPE)
    vel8 = vel8.reshape(16, STRIPE)
    b1r = b1.reshape(1, HIDDEN)
    b2r = b2.reshape(1, HALF)
    b3r = b3.reshape(1, 3)

    grid_spec = pltpu.PrefetchScalarGridSpec(
        num_scalar_prefetch=1,
        grid=(GRID,),
        in_specs=[
            pl.BlockSpec((T_FUT, 4), lambda g, i: (0, 0)),
            pl.BlockSpec((TWO_T, STRIPE), lambda g, i: (0, 0)),
            pl.BlockSpec((16, STRIPE), lambda g, i: (0, 0)),
            pl.BlockSpec((8, STRIPE), lambda g, i: (0, 0)),
            pl.BlockSpec((TWO_T, D_ENC), lambda g, i: (0, 0)),
            pl.BlockSpec((D_ENC, MODES6 * T_FUT * 2), lambda g, i: (0, 0)),
            pl.BlockSpec((D_ENC, MODES6 * T_FUT), lambda g, i: (0, 0)),
            pl.BlockSpec((3, CHUNK), lambda g, i: (0, g)),
            pl.BlockSpec((1, CHUNK), lambda g, i: (0, g)),
            pl.BlockSpec((CHUNK, HALF), lambda g, i: (g, 0)),
            pl.BlockSpec((1, HALF), lambda g, i: (0, 0)),
            pl.BlockSpec((HALF, 3), lambda g, i: (0, 0)),
            pl.BlockSpec((1, 3), lambda g, i: (0, 0)),
        ],
        out_specs=[
            pl.BlockSpec((T_FUT, 3), lambda g, i: (0, 0)),
            pl.BlockSpec((T_FUT, 3), lambda g, i: (0, 0)),
        ],
        scratch_shapes=[
            pltpu.VMEM((T_FUT, 3), jnp.float32),
            pltpu.VMEM((T_FUT, HALF), jnp.float32),
        ],
    )
    mean, std = pl.pallas_call(
        _policy_kernel,
        grid_spec=grid_spec,
        out_shape=[
            jax.ShapeDtypeStruct((T_FUT, 3), jnp.float32),
            jax.ShapeDtypeStruct((T_FUT, 3), jnp.float32),
        ],
    )(idx, states, pos8, vel8, head8,
      W_enc, W_dec_pos, W_dec_head,
      W1, b1r, W2, b2r, W3, b3r)
    return (mean, std)


# speedup vs baseline: 13.6471x; 1.0350x over previous
import jax
import jax.numpy as jnp
from jax import lax
from jax.experimental import pallas as pl
from jax.experimental.pallas import tpu as pltpu
from jax.experimental.pallas import tpu_sc as plsc

T_HIST = 50
T_FUT = 60
T_TOT = T_HIST + T_FUT
TWO_T = 2 * T_TOT
D_ENC = 512
MODES6 = 6
HIDDEN = 4096
HALF = HIDDEN // 2
CHUNK = 512
GRID = HIDDEN // CHUNK
STRIPE = 128
TSUB = 48

_TN = (((0,), (0,)), ((), ()))
_TT = (((0,), (1,)), ((), ()))


def _sc_gather(idx_hbm, pos_hbm, vel_hbm, head_hbm,
               pos_out, vel_out, head_out, idx_s):
    c = lax.axis_index("c")
    pltpu.sync_copy(idx_hbm, idx_s)
    ib = (idx_s[0] // STRIPE) * STRIPE

    @pl.when(c == 0)
    def _gather_x_head():
        pltpu.sync_copy(pos_hbm.at[pl.ds(0, 1), :, pl.ds(ib, STRIPE)],
                        pos_out.at[pl.ds(0, 1)])
        pltpu.sync_copy(head_hbm.at[pl.ds(TSUB, 8), pl.ds(ib, STRIPE)],
                        head_out)

    @pl.when(c == 1)
    def _gather_y_vel():
        pltpu.sync_copy(pos_hbm.at[pl.ds(1, 1), :, pl.ds(ib, STRIPE)],
                        pos_out.at[pl.ds(1, 1)])
        pltpu.sync_copy(
            vel_hbm.at[pl.ds(0, 2), pl.ds(TSUB, 8), pl.ds(ib, STRIPE)],
            vel_out)


def _gather_stripes(idx, pos_t, vel_t, head_t):
    mesh = plsc.ScalarSubcoreMesh(axis_name="c", num_cores=2)
    f32 = jnp.float32
    return pl.kernel(
        _sc_gather,
        out_type=[
            jax.ShapeDtypeStruct((2, T_TOT, STRIPE), f32),
            jax.ShapeDtypeStruct((2, 8, STRIPE), f32),
            jax.ShapeDtypeStruct((8, STRIPE), f32),
        ],
        mesh=mesh,
        scratch_types=[
            pltpu.SMEM((1,), jnp.int32),
        ],
    )(idx, pos_t, vel_t, head_t)


def _policy_kernel(idx_ref, states_ref, pos8_ref, vel8_ref, head8_ref,
                   wenc_ref, wdp_ref, wdh_ref,
                   w1_ref, b1_ref, w2_ref, b2_ref, w3_ref, b3_ref,
                   mean_ref, std_ref, act_s, acc_s):
    g = pl.program_id(0)

    @pl.when(g == 0)
    def _build_action():
        r = jnp.remainder(idx_ref[0], STRIPE)
        lane_p = jax.lax.broadcasted_iota(jnp.int32, (TWO_T, STRIPE), 1)
        cols = jnp.sum(jnp.where(lane_p == r, pos8_ref[...], 0.0),
                       axis=1, keepdims=True)
        x_col = cols[0:T_TOT, :]
        y_col = cols[T_TOT:TWO_T, :]
        lane_v = jax.lax.broadcasted_iota(jnp.int32, (16, STRIPE), 1)
        v49c = jnp.sum(jnp.where(lane_v == r, vel8_ref[...], 0.0),
                       axis=1, keepdims=True)
        lane_h = jax.lax.broadcasted_iota(jnp.int32, (8, STRIPE), 1)
        thc = jnp.sum(jnp.where(lane_h == r, head8_ref[...], 0.0),
                      axis=1, keepdims=True)
        theta = thc[T_HIST - 1 - TSUB, 0]
        ox = x_col[T_HIST - 1, 0]
        oy = y_col[T_HIST - 1, 0]
        v49x = v49c[(T_HIST - 1 - TSUB), 0]
        v49y = v49c[8 + (T_HIST - 1 - TSUB), 0]
        st = states_ref[...]
        st_pad = jnp.pad(st, ((T_HIST, 0), (0, 0)))
        t110 = jax.lax.broadcasted_iota(jnp.int32, (T_TOT, 1), 0)
        fut = t110 >= T_HIST
        x_col = jnp.where(fut, st_pad[:, 0:1], x_col)
        y_col = jnp.where(fut, st_pad[:, 1:2], y_col)
        r220 = jax.lax.broadcasted_iota(jnp.int32, (TWO_T, T_TOT), 0)
        c110 = jax.lax.broadcasted_iota(jnp.int32, (TWO_T, T_TOT), 1)
        px = (r220 == 2 * c110).astype(jnp.float32)
        py = (r220 == 2 * c110 + 1).astype(jnp.float32)
        xy = (jnp.dot(px, x_col, preferred_element_type=jnp.float32)
              + jnp.dot(py, y_col, preferred_element_type=jnp.float32))
        feat = jnp.tanh(jax.lax.dot_general(
            xy, wenc_ref[...], _TN, preferred_element_type=jnp.float32))
        dec = jnp.dot(feat, wdp_ref[...],
                      preferred_element_type=jnp.float32)
        dech = jnp.dot(feat, wdh_ref[...],
                       preferred_element_type=jnp.float32)
        rp = jax.lax.broadcasted_iota(jnp.int32, (MODES6 * T_FUT * 2, T_FUT), 0)
        cp = jax.lax.broadcasted_iota(jnp.int32, (MODES6 * T_FUT * 2, T_FUT), 1)
        sel_x = (rp == 2 * cp).astype(jnp.float32)
        sel_y = (rp == 2 * cp + 1).astype(jnp.float32)
        rh = jax.lax.broadcasted_iota(jnp.int32, (MODES6 * T_FUT, T_FUT), 0)
        ch = jax.lax.broadcasted_iota(jnp.int32, (MODES6 * T_FUT, T_FUT), 1)
        sel_h = (rh == ch).astype(jnp.float32)
        lpx = jax.lax.dot_general(sel_x, dec, _TT,
                                  preferred_element_type=jnp.float32)
        lpy = jax.lax.dot_general(sel_y, dec, _TT,
                                  preferred_element_type=jnp.float32)
        lh = jax.lax.dot_general(sel_h, dech, _TT,
                                 preferred_element_type=jnp.float32)
        c, s = jnp.cos(theta), jnp.sin(theta)
        npx = lpx * c - lpy * s + ox
        npy = lpx * s + lpy * c + oy
        t60 = jax.lax.broadcasted_iota(jnp.int32, (T_FUT, 1), 0)
        z1 = jnp.zeros((1, 1), jnp.float32)
        px_prev = jnp.concatenate([z1, npx[:-1, :]], axis=0)
        py_prev = jnp.concatenate([z1, npy[:-1, :]], axis=0)
        vx = jnp.where(t60 == 0, npx - ox, (npx - px_prev) * 10.0)
        vy = jnp.where(t60 == 0, npy - oy, (npy - py_prev) * 10.0)
        vx_prev = jnp.concatenate([z1, vx[:-1, :]], axis=0)
        vy_prev = jnp.concatenate([z1, vy[:-1, :]], axis=0)
        ax = jnp.where(t60 == 0, vx - v49x, (vx - vx_prev) * 10.0)
        ay = jnp.where(t60 == 0, vy - v49y, (vy - vy_prev) * 10.0)
        two_pi = 2.0 * jnp.pi
        hd = lh + theta
        hd = (hd + jnp.pi) % two_pi - jnp.pi
        act_s[...] = jnp.concatenate([hd, ax, ay], axis=1)
        acc_s[...] = jnp.zeros_like(acc_s)

    hd = act_s[:, 0:1]
    ax = act_s[:, 1:2]
    ay = act_s[:, 2:3]
    h1 = (hd * w1_ref[0:1, :] + ax * w1_ref[1:2, :] + ay * w1_ref[2:3, :]
          + b1_ref[...])
    h1 = jnp.maximum(h1, 0.0)
    acc_s[...] += jnp.dot(h1, w2_ref[...], preferred_element_type=jnp.float32)

    @pl.when(g == GRID - 1)
    def _finalize():
        h2 = acc_s[...] + b2_ref[...]
        y = jnp.dot(h2, w3_ref[...], preferred_element_type=jnp.float32) \
            + b3_ref[...]
        mean_ref[...] = jnp.tanh(y)
        std_ref[...] = jnp.log1p(jnp.exp(-jnp.abs(y))) + jnp.maximum(y, 0.0) \
            + 1e-8


def kernel(states, position, velocity, heading, predict_mask, agent_index,
           W_enc, W_dec_pos, W_dec_head, W1, b1, W2, b2, W3, b3):
    del predict_mask
    idx = jnp.asarray(agent_index, jnp.int32).reshape((1,))
    pos_t = jnp.transpose(position, (2, 1, 0))
    vel_t = jnp.transpose(velocity, (2, 1, 0))
    head_t = jnp.transpose(heading, (1, 0))
    pos8, vel8, head8 = _gather_stripes(idx, pos_t, vel_t, head_t)
    pos8 = pos8.reshape(TWO_T, STRIPE)
    vel8 = vel8.reshape(16, STRIPE)
    b1r = b1.reshape(1, HIDDEN)
    b2r = b2.reshape(1, HALF)
    b3r = b3.reshape(1, 3)

    grid_spec = pltpu.PrefetchScalarGridSpec(
        num_scalar_prefetch=1,
        grid=(GRID,),
        in_specs=[
            pl.BlockSpec((T_FUT, 4), lambda g, i: (0, 0)),
            pl.BlockSpec((TWO_T, STRIPE), lambda g, i: (0, 0)),
            pl.BlockSpec((16, STRIPE), lambda g, i: (0, 0)),
            pl.BlockSpec((8, STRIPE), lambda g, i: (0, 0)),
            pl.BlockSpec((TWO_T, D_ENC), lambda g, i: (0, 0)),
            pl.BlockSpec((D_ENC, MODES6 * T_FUT * 2), lambda g, i: (0, 0)),
            pl.BlockSpec((D_ENC, MODES6 * T_FUT), lambda g, i: (0, 0)),
            pl.BlockSpec((3, CHUNK), lambda g, i: (0, g)),
            pl.BlockSpec((1, CHUNK), lambda g, i: (0, g)),
            pl.BlockSpec((CHUNK, HALF), lambda g, i: (g, 0)),
            pl.BlockSpec((1, HALF), lambda g, i: (0, 0)),
            pl.BlockSpec((HALF, 3), lambda g, i: (0, 0)),
            pl.BlockSpec((1, 3), lambda g, i: (0, 0)),
        ],
        out_specs=[
            pl.BlockSpec((T_FUT, 3), lambda g, i: (0, 0)),
            pl.BlockSpec((T_FUT, 3), lambda g, i: (0, 0)),
        ],
        scratch_shapes=[
            pltpu.VMEM((T_FUT, 3), jnp.float32),
            pltpu.VMEM((T_FUT, HALF), jnp.float32),
        ],
    )
    mean, std = pl.pallas_call(
        _policy_kernel,
        grid_spec=grid_spec,
        out_shape=[
            jax.ShapeDtypeStruct((T_FUT, 3), jnp.float32),
            jax.ShapeDtypeStruct((T_FUT, 3), jnp.float32),
        ],
    )(idx, states, pos8, vel8, head8,
      W_enc, W_dec_pos, W_dec_head,
      W1, b1r, W2, b2r, W3, b3r)
    return (mean, std)


# speedup vs baseline: 14.1276x; 1.0352x over previous
import jax
import jax.numpy as jnp
from jax import lax
from jax.experimental import pallas as pl
from jax.experimental.pallas import tpu as pltpu
from jax.experimental.pallas import tpu_sc as plsc

T_HIST = 50
T_FUT = 60
T_TOT = T_HIST + T_FUT
TWO_T = 2 * T_TOT
D_ENC = 512
MODES6 = 6
HIDDEN = 4096
HALF = HIDDEN // 2
CHUNK = 1024
GRID = HIDDEN // CHUNK
STRIPE = 128
TSUB = 48

_TN = (((0,), (0,)), ((), ()))
_TT = (((0,), (1,)), ((), ()))


def _sc_gather(idx_hbm, pos_hbm, vel_hbm, head_hbm,
               pos_out, vel_out, head_out, idx_s):
    c = lax.axis_index("c")
    pltpu.sync_copy(idx_hbm, idx_s)
    ib = (idx_s[0] // STRIPE) * STRIPE

    @pl.when(c == 0)
    def _gather_x_head():
        pltpu.sync_copy(pos_hbm.at[pl.ds(0, 1), :, pl.ds(ib, STRIPE)],
                        pos_out.at[pl.ds(0, 1)])
        pltpu.sync_copy(head_hbm.at[pl.ds(TSUB, 8), pl.ds(ib, STRIPE)],
                        head_out)

    @pl.when(c == 1)
    def _gather_y_vel():
        pltpu.sync_copy(pos_hbm.at[pl.ds(1, 1), :, pl.ds(ib, STRIPE)],
                        pos_out.at[pl.ds(1, 1)])
        pltpu.sync_copy(
            vel_hbm.at[pl.ds(0, 2), pl.ds(TSUB, 8), pl.ds(ib, STRIPE)],
            vel_out)


def _gather_stripes(idx, pos_t, vel_t, head_t):
    mesh = plsc.ScalarSubcoreMesh(axis_name="c", num_cores=2)
    f32 = jnp.float32
    return pl.kernel(
        _sc_gather,
        out_type=[
            jax.ShapeDtypeStruct((2, T_TOT, STRIPE), f32),
            jax.ShapeDtypeStruct((2, 8, STRIPE), f32),
            jax.ShapeDtypeStruct((8, STRIPE), f32),
        ],
        mesh=mesh,
        scratch_types=[
            pltpu.SMEM((1,), jnp.int32),
        ],
    )(idx, pos_t, vel_t, head_t)


def _policy_kernel(idx_ref, states_ref, pos8_ref, vel8_ref, head8_ref,
                   wenc_ref, wdp_ref, wdh_ref,
                   w1_ref, b1_ref, w2_ref, b2_ref, w3_ref, b3_ref,
                   mean_ref, std_ref, act_s, acc_s):
    g = pl.program_id(0)

    @pl.when(g == 0)
    def _build_action():
        r = jnp.remainder(idx_ref[0], STRIPE)
        lane_p = jax.lax.broadcasted_iota(jnp.int32, (TWO_T, STRIPE), 1)
        cols = jnp.sum(jnp.where(lane_p == r, pos8_ref[...], 0.0),
                       axis=1, keepdims=True)
        x_col = cols[0:T_TOT, :]
        y_col = cols[T_TOT:TWO_T, :]
        lane_v = jax.lax.broadcasted_iota(jnp.int32, (16, STRIPE), 1)
        v49c = jnp.sum(jnp.where(lane_v == r, vel8_ref[...], 0.0),
                       axis=1, keepdims=True)
        lane_h = jax.lax.broadcasted_iota(jnp.int32, (8, STRIPE), 1)
        thc = jnp.sum(jnp.where(lane_h == r, head8_ref[...], 0.0),
                      axis=1, keepdims=True)
        theta = thc[T_HIST - 1 - TSUB, 0]
        ox = x_col[T_HIST - 1, 0]
        oy = y_col[T_HIST - 1, 0]
        v49x = v49c[(T_HIST - 1 - TSUB), 0]
        v49y = v49c[8 + (T_HIST - 1 - TSUB), 0]
        st = states_ref[...]
        st_pad = jnp.pad(st, ((T_HIST, 0), (0, 0)))
        t110 = jax.lax.broadcasted_iota(jnp.int32, (T_TOT, 1), 0)
        fut = t110 >= T_HIST
        x_col = jnp.where(fut, st_pad[:, 0:1], x_col)
        y_col = jnp.where(fut, st_pad[:, 1:2], y_col)
        r220 = jax.lax.broadcasted_iota(jnp.int32, (TWO_T, T_TOT), 0)
        c110 = jax.lax.broadcasted_iota(jnp.int32, (TWO_T, T_TOT), 1)
        px = (r220 == 2 * c110).astype(jnp.float32)
        py = (r220 == 2 * c110 + 1).astype(jnp.float32)
        xy = (jnp.dot(px, x_col, preferred_element_type=jnp.float32)
              + jnp.dot(py, y_col, preferred_element_type=jnp.float32))
        feat = jnp.tanh(jax.lax.dot_general(
            xy, wenc_ref[...], _TN, preferred_element_type=jnp.float32))
        dec = jnp.dot(feat, wdp_ref[...],
                      preferred_element_type=jnp.float32)
        dech = jnp.dot(feat, wdh_ref[...],
                       preferred_element_type=jnp.float32)
        rp = jax.lax.broadcasted_iota(jnp.int32, (MODES6 * T_FUT * 2, T_FUT), 0)
        cp = jax.lax.broadcasted_iota(jnp.int32, (MODES6 * T_FUT * 2, T_FUT), 1)
        sel_x = (rp == 2 * cp).astype(jnp.float32)
        sel_y = (rp == 2 * cp + 1).astype(jnp.float32)
        rh = jax.lax.broadcasted_iota(jnp.int32, (MODES6 * T_FUT, T_FUT), 0)
        ch = jax.lax.broadcasted_iota(jnp.int32, (MODES6 * T_FUT, T_FUT), 1)
        sel_h = (rh == ch).astype(jnp.float32)
        lpx = jax.lax.dot_general(sel_x, dec, _TT,
                                  preferred_element_type=jnp.float32)
        lpy = jax.lax.dot_general(sel_y, dec, _TT,
                                  preferred_element_type=jnp.float32)
        lh = jax.lax.dot_general(sel_h, dech, _TT,
                                 preferred_element_type=jnp.float32)
        c, s = jnp.cos(theta), jnp.sin(theta)
        npx = lpx * c - lpy * s + ox
        npy = lpx * s + lpy * c + oy
        t60 = jax.lax.broadcasted_iota(jnp.int32, (T_FUT, 1), 0)
        z1 = jnp.zeros((1, 1), jnp.float32)
        px_prev = jnp.concatenate([z1, npx[:-1, :]], axis=0)
        py_prev = jnp.concatenate([z1, npy[:-1, :]], axis=0)
        vx = jnp.where(t60 == 0, npx - ox, (npx - px_prev) * 10.0)
        vy = jnp.where(t60 == 0, npy - oy, (npy - py_prev) * 10.0)
        vx_prev = jnp.concatenate([z1, vx[:-1, :]], axis=0)
        vy_prev = jnp.concatenate([z1, vy[:-1, :]], axis=0)
        ax = jnp.where(t60 == 0, vx - v49x, (vx - vx_prev) * 10.0)
        ay = jnp.where(t60 == 0, vy - v49y, (vy - vy_prev) * 10.0)
        two_pi = 2.0 * jnp.pi
        hd = lh + theta
        hd = (hd + jnp.pi) % two_pi - jnp.pi
        act_s[...] = jnp.concatenate([hd, ax, ay], axis=1)
        acc_s[...] = jnp.zeros_like(acc_s)

    hd = act_s[:, 0:1]
    ax = act_s[:, 1:2]
    ay = act_s[:, 2:3]
    h1 = (hd * w1_ref[0:1, :] + ax * w1_ref[1:2, :] + ay * w1_ref[2:3, :]
          + b1_ref[...])
    h1 = jnp.maximum(h1, 0.0)
    acc_s[...] += jnp.dot(h1, w2_ref[...], preferred_element_type=jnp.float32)

    @pl.when(g == GRID - 1)
    def _finalize():
        h2 = acc_s[...] + b2_ref[...]
        y = jnp.dot(h2, w3_ref[...], preferred_element_type=jnp.float32) \
            + b3_ref[...]
        mean_ref[...] = jnp.tanh(y)
        std_ref[...] = jnp.log1p(jnp.exp(-jnp.abs(y))) + jnp.maximum(y, 0.0) \
            + 1e-8


def kernel(states, position, velocity, heading, predict_mask, agent_index,
           W_enc, W_dec_pos, W_dec_head, W1, b1, W2, b2, W3, b3):
    del predict_mask
    idx = jnp.asarray(agent_index, jnp.int32).reshape((1,))
    pos_t = jnp.transpose(position, (2, 1, 0))
    vel_t = jnp.transpose(velocity, (2, 1, 0))
    head_t = jnp.transpose(heading, (1, 0))
    pos8, vel8, head8 = _gather_stripes(idx, pos_t, vel_t, head_t)
    pos8 = pos8.reshape(TWO_T, STRIPE)
    vel8 = vel8.reshape(16, STRIPE)
    b1r = b1.reshape(1, HIDDEN)
    b2r = b2.reshape(1, HALF)
    b3r = b3.reshape(1, 3)

    grid_spec = pltpu.PrefetchScalarGridSpec(
        num_scalar_prefetch=1,
        grid=(GRID,),
        in_specs=[
            pl.BlockSpec((T_FUT, 4), lambda g, i: (0, 0)),
            pl.BlockSpec((TWO_T, STRIPE), lambda g, i: (0, 0)),
            pl.BlockSpec((16, STRIPE), lambda g, i: (0, 0)),
            pl.BlockSpec((8, STRIPE), lambda g, i: (0, 0)),
            pl.BlockSpec((TWO_T, D_ENC), lambda g, i: (0, 0)),
            pl.BlockSpec((D_ENC, MODES6 * T_FUT * 2), lambda g, i: (0, 0)),
            pl.BlockSpec((D_ENC, MODES6 * T_FUT), lambda g, i: (0, 0)),
            pl.BlockSpec((3, CHUNK), lambda g, i: (0, g)),
            pl.BlockSpec((1, CHUNK), lambda g, i: (0, g)),
            pl.BlockSpec((CHUNK, HALF), lambda g, i: (g, 0)),
            pl.BlockSpec((1, HALF), lambda g, i: (0, 0)),
            pl.BlockSpec((HALF, 3), lambda g, i: (0, 0)),
            pl.BlockSpec((1, 3), lambda g, i: (0, 0)),
        ],
        out_specs=[
            pl.BlockSpec((T_FUT, 3), lambda g, i: (0, 0)),
            pl.BlockSpec((T_FUT, 3), lambda g, i: (0, 0)),
        ],
        scratch_shapes=[
            pltpu.VMEM((T_FUT, 3), jnp.float32),
            pltpu.VMEM((T_FUT, HALF), jnp.float32),
        ],
    )
    mean, std = pl.pallas_call(
        _policy_kernel,
        grid_spec=grid_spec,
        out_shape=[
            jax.ShapeDtypeStruct((T_FUT, 3), jnp.float32),
            jax.ShapeDtypeStruct((T_FUT, 3), jnp.float32),
        ],
    )(idx, states, pos8, vel8, head8,
      W_enc, W_dec_pos, W_dec_head,
      W1, b1r, W2, b2r, W3, b3r)
    return (mean, std)


# speedup vs baseline: 14.1528x; 1.0018x over previous
import jax
import jax.numpy as jnp
from jax import lax
from jax.experimental import pallas as pl
from jax.experimental.pallas import tpu as pltpu
from jax.experimental.pallas import tpu_sc as plsc

T_HIST = 50
T_FUT = 60
T_TOT = T_HIST + T_FUT
TWO_T = 2 * T_TOT
D_ENC = 512
MODES6 = 6
HIDDEN = 4096
HALF = HIDDEN // 2
CHUNK = 2048
GRID = HIDDEN // CHUNK
STRIPE = 128
TSUB = 48

_TN = (((0,), (0,)), ((), ()))
_TT = (((0,), (1,)), ((), ()))


def _sc_gather(idx_hbm, pos_hbm, vel_hbm, head_hbm,
               pos_out, vel_out, head_out, idx_s):
    c = lax.axis_index("c")
    pltpu.sync_copy(idx_hbm, idx_s)
    ib = (idx_s[0] // STRIPE) * STRIPE

    @pl.when(c == 0)
    def _gather_x_head():
        pltpu.sync_copy(pos_hbm.at[pl.ds(0, 1), :, pl.ds(ib, STRIPE)],
                        pos_out.at[pl.ds(0, 1)])
        pltpu.sync_copy(head_hbm.at[pl.ds(TSUB, 8), pl.ds(ib, STRIPE)],
                        head_out)

    @pl.when(c == 1)
    def _gather_y_vel():
        pltpu.sync_copy(pos_hbm.at[pl.ds(1, 1), :, pl.ds(ib, STRIPE)],
                        pos_out.at[pl.ds(1, 1)])
        pltpu.sync_copy(
            vel_hbm.at[pl.ds(0, 2), pl.ds(TSUB, 8), pl.ds(ib, STRIPE)],
            vel_out)


def _gather_stripes(idx, pos_t, vel_t, head_t):
    mesh = plsc.ScalarSubcoreMesh(axis_name="c", num_cores=2)
    f32 = jnp.float32
    return pl.kernel(
        _sc_gather,
        out_type=[
            jax.ShapeDtypeStruct((2, T_TOT, STRIPE), f32),
            jax.ShapeDtypeStruct((2, 8, STRIPE), f32),
            jax.ShapeDtypeStruct((8, STRIPE), f32),
        ],
        mesh=mesh,
        scratch_types=[
            pltpu.SMEM((1,), jnp.int32),
        ],
    )(idx, pos_t, vel_t, head_t)


def _policy_kernel(idx_ref, states_ref, pos8_ref, vel8_ref, head8_ref,
                   wenc_ref, wdp_ref, wdh_ref,
                   w1_ref, b1_ref, w2_ref, b2_ref, w3_ref, b3_ref,
                   mean_ref, std_ref, act_s, acc_s):
    g = pl.program_id(0)

    @pl.when(g == 0)
    def _build_action():
        r = jnp.remainder(idx_ref[0], STRIPE)
        lane_p = jax.lax.broadcasted_iota(jnp.int32, (TWO_T, STRIPE), 1)
        cols = jnp.sum(jnp.where(lane_p == r, pos8_ref[...], 0.0),
                       axis=1, keepdims=True)
        x_col = cols[0:T_TOT, :]
        y_col = cols[T_TOT:TWO_T, :]
        lane_v = jax.lax.broadcasted_iota(jnp.int32, (16, STRIPE), 1)
        v49c = jnp.sum(jnp.where(lane_v == r, vel8_ref[...], 0.0),
                       axis=1, keepdims=True)
        lane_h = jax.lax.broadcasted_iota(jnp.int32, (8, STRIPE), 1)
        thc = jnp.sum(jnp.where(lane_h == r, head8_ref[...], 0.0),
                      axis=1, keepdims=True)
        theta = thc[T_HIST - 1 - TSUB, 0]
        ox = x_col[T_HIST - 1, 0]
        oy = y_col[T_HIST - 1, 0]
        v49x = v49c[(T_HIST - 1 - TSUB), 0]
        v49y = v49c[8 + (T_HIST - 1 - TSUB), 0]
        st = states_ref[...]
        st_pad = jnp.pad(st, ((T_HIST, 0), (0, 0)))
        t110 = jax.lax.broadcasted_iota(jnp.int32, (T_TOT, 1), 0)
        fut = t110 >= T_HIST
        x_col = jnp.where(fut, st_pad[:, 0:1], x_col)
        y_col = jnp.where(fut, st_pad[:, 1:2], y_col)
        r220 = jax.lax.broadcasted_iota(jnp.int32, (TWO_T, T_TOT), 0)
        c110 = jax.lax.broadcasted_iota(jnp.int32, (TWO_T, T_TOT), 1)
        px = (r220 == 2 * c110).astype(jnp.float32)
        py = (r220 == 2 * c110 + 1).astype(jnp.float32)
        xy = (jnp.dot(px, x_col, preferred_element_type=jnp.float32)
              + jnp.dot(py, y_col, preferred_element_type=jnp.float32))
        feat = jnp.tanh(jax.lax.dot_general(
            xy, wenc_ref[...], _TN, preferred_element_type=jnp.float32))
        dec = jnp.dot(feat, wdp_ref[...],
                      preferred_element_type=jnp.float32)
        dech = jnp.dot(feat, wdh_ref[...],
                       preferred_element_type=jnp.float32)
        rp = jax.lax.broadcasted_iota(jnp.int32, (MODES6 * T_FUT * 2, T_FUT), 0)
        cp = jax.lax.broadcasted_iota(jnp.int32, (MODES6 * T_FUT * 2, T_FUT), 1)
        sel_x = (rp == 2 * cp).astype(jnp.float32)
        sel_y = (rp == 2 * cp + 1).astype(jnp.float32)
        rh = jax.lax.broadcasted_iota(jnp.int32, (MODES6 * T_FUT, T_FUT), 0)
        ch = jax.lax.broadcasted_iota(jnp.int32, (MODES6 * T_FUT, T_FUT), 1)
        sel_h = (rh == ch).astype(jnp.float32)
        lpx = jax.lax.dot_general(sel_x, dec, _TT,
                                  preferred_element_type=jnp.float32)
        lpy = jax.lax.dot_general(sel_y, dec, _TT,
                                  preferred_element_type=jnp.float32)
        lh = jax.lax.dot_general(sel_h, dech, _TT,
                                 preferred_element_type=jnp.float32)
        c, s = jnp.cos(theta), jnp.sin(theta)
        npx = lpx * c - lpy * s + ox
        npy = lpx * s + lpy * c + oy
        t60 = jax.lax.broadcasted_iota(jnp.int32, (T_FUT, 1), 0)
        z1 = jnp.zeros((1, 1), jnp.float32)
        px_prev = jnp.concatenate([z1, npx[:-1, :]], axis=0)
        py_prev = jnp.concatenate([z1, npy[:-1, :]], axis=0)
        vx = jnp.where(t60 == 0, npx - ox, (npx - px_prev) * 10.0)
        vy = jnp.where(t60 == 0, npy - oy, (npy - py_prev) * 10.0)
        vx_prev = jnp.concatenate([z1, vx[:-1, :]], axis=0)
        vy_prev = jnp.concatenate([z1, vy[:-1, :]], axis=0)
        ax = jnp.where(t60 == 0, vx - v49x, (vx - vx_prev) * 10.0)
        ay = jnp.where(t60 == 0, vy - v49y, (vy - vy_prev) * 10.0)
        two_pi = 2.0 * jnp.pi
        hd = lh + theta
        hd = (hd + jnp.pi) % two_pi - jnp.pi
        act_s[...] = jnp.concatenate([hd, ax, ay], axis=1)
        acc_s[...] = jnp.zeros_like(acc_s)

    hd = act_s[:, 0:1]
    ax = act_s[:, 1:2]
    ay = act_s[:, 2:3]
    h1 = (hd * w1_ref[0:1, :] + ax * w1_ref[1:2, :] + ay * w1_ref[2:3, :]
          + b1_ref[...])
    h1 = jnp.maximum(h1, 0.0)
    acc_s[...] += jnp.dot(h1, w2_ref[...], preferred_element_type=jnp.float32)

    @pl.when(g == GRID - 1)
    def _finalize():
        h2 = acc_s[...] + b2_ref[...]
        y = jnp.dot(h2, w3_ref[...], preferred_element_type=jnp.float32) \
            + b3_ref[...]
        mean_ref[...] = jnp.tanh(y)
        std_ref[...] = jnp.log1p(jnp.exp(-jnp.abs(y))) + jnp.maximum(y, 0.0) \
            + 1e-8


def kernel(states, position, velocity, heading, predict_mask, agent_index,
           W_enc, W_dec_pos, W_dec_head, W1, b1, W2, b2, W3, b3):
    del predict_mask
    idx = jnp.asarray(agent_index, jnp.int32).reshape((1,))
    pos_t = jnp.transpose(position, (2, 1, 0))
    vel_t = jnp.transpose(velocity, (2, 1, 0))
    head_t = jnp.transpose(heading, (1, 0))
    pos8, vel8, head8 = _gather_stripes(idx, pos_t, vel_t, head_t)
    pos8 = pos8.reshape(TWO_T, STRIPE)
    vel8 = vel8.reshape(16, STRIPE)
    b1r = b1.reshape(1, HIDDEN)
    b2r = b2.reshape(1, HALF)
    b3r = b3.reshape(1, 3)

    grid_spec = pltpu.PrefetchScalarGridSpec(
        num_scalar_prefetch=1,
        grid=(GRID,),
        in_specs=[
            pl.BlockSpec((T_FUT, 4), lambda g, i: (0, 0)),
            pl.BlockSpec((TWO_T, STRIPE), lambda g, i: (0, 0)),
            pl.BlockSpec((16, STRIPE), lambda g, i: (0, 0)),
            pl.BlockSpec((8, STRIPE), lambda g, i: (0, 0)),
            pl.BlockSpec((TWO_T, D_ENC), lambda g, i: (0, 0)),
            pl.BlockSpec((D_ENC, MODES6 * T_FUT * 2), lambda g, i: (0, 0)),
            pl.BlockSpec((D_ENC, MODES6 * T_FUT), lambda g, i: (0, 0)),
            pl.BlockSpec((3, CHUNK), lambda g, i: (0, g)),
            pl.BlockSpec((1, CHUNK), lambda g, i: (0, g)),
            pl.BlockSpec((CHUNK, HALF), lambda g, i: (g, 0)),
            pl.BlockSpec((1, HALF), lambda g, i: (0, 0)),
            pl.BlockSpec((HALF, 3), lambda g, i: (0, 0)),
            pl.BlockSpec((1, 3), lambda g, i: (0, 0)),
        ],
        out_specs=[
            pl.BlockSpec((T_FUT, 3), lambda g, i: (0, 0)),
            pl.BlockSpec((T_FUT, 3), lambda g, i: (0, 0)),
        ],
        scratch_shapes=[
            pltpu.VMEM((T_FUT, 3), jnp.float32),
            pltpu.VMEM((T_FUT, HALF), jnp.float32),
        ],
    )
    mean, std = pl.pallas_call(
        _policy_kernel,
        grid_spec=grid_spec,
        out_shape=[
            jax.ShapeDtypeStruct((T_FUT, 3), jnp.float32),
            jax.ShapeDtypeStruct((T_FUT, 3), jnp.float32),
        ],
    )(idx, states, pos8, vel8, head8,
      W_enc, W_dec_pos, W_dec_head,
      W1, b1r, W2, b2r, W3, b3r)
    return (mean, std)
